# baseline (speedup 1.0000x reference)
import jax
import jax.numpy as jnp
from jax.experimental import pallas as pl
from jax.experimental.pallas import tpu as pltpu

_N_EVENTS = 10000
_NSPLITS = 5
_NF_IN = 4
_NF = 32
_NG = 16
_BE = 400
_NNODES = 2 ** (_NSPLITS + 1) - 1


def _mm(x, w, b):
    return jnp.maximum(jnp.dot(x, w, preferred_element_type=jnp.float32) + b, 0.0)


def _tree_body(rv_ref, *refs):
    out_ref = refs[-1]
    (pw0, pb0, pw1, pb1, qw0, qb0, qw1, qb1,
     bw0, bb0, bw1, bb1, bw2, bb2,
     mw0, mb0, mw1, mb1, mw2, mb2,
     uw0, ub0, uw1, ub1, uw2, ub2) = [r[...] for r in refs[:-1]]
    be = rv_ref.shape[0]
    xs = [rv_ref[...].reshape(1, be, _NF)]
    for inx in range(_NSPLITS):
        na = 2 ** (inx + 1) - 1
        xa = jnp.concatenate([xs[l].reshape(-1, _NF) for l in range(inx + 1)],
                             axis=0)
        h = _mm(_mm(xa, pw0, pb0), pw1, pb1)
        pooled = h.reshape(na, be, _NG).sum(axis=0) * (1.0 / na)
        g = _mm(_mm(pooled, qw0, qb0), qw1, qb1)
        nl = 2 ** inx
        leaf = xs[inx].reshape(nl * be, _NF)
        gl = jnp.broadcast_to(g, (nl, be, _NG)).reshape(nl * be, _NG)
        c = _mm(_mm(_mm(jnp.concatenate([leaf, gl], axis=1), bw0, bb0),
                    bw1, bb1), bw2, bb2)
        c0 = c[:, :_NF].reshape(nl, 1, be, _NF)
        c1 = c[:, _NF:].reshape(nl, 1, be, _NF)
        xs.append(jnp.concatenate([c0, c1], axis=1).reshape(2 * nl, be, _NF))
        ga = jnp.broadcast_to(g, (na, be, _NG)).reshape(na * be, _NG)
        m = _mm(_mm(_mm(jnp.concatenate([xa, ga], axis=1), mw0, mb0),
                    mw1, mb1), mw2, mb2)
        m = m.reshape(na, be, _NF)
        s_lvl = [m[0:1]]
        row = 1
        for l in range(1, inx + 1):
            cnt = 2 ** l
            s_lvl.append(jnp.repeat(s_lvl[l - 1], 2, axis=0) + m[row:row + cnt])
            row += cnt
        agg = [jnp.zeros((1, be, _NF), jnp.float32)]
        for l in range(1, inx + 2):
            agg.append(jnp.repeat(s_lvl[l - 1], 2, axis=0))
        ua = jnp.concatenate([
            jnp.concatenate(
                [xs[l], agg[l], jnp.broadcast_to(g, (2 ** l, be, _NG))], axis=2
            ).reshape(-1, 2 * _NF + _NG)
            for l in range(inx + 2)
        ], axis=0)
        u = _mm(_mm(_mm(ua, uw0, ub0), uw1, ub1), uw2, ub2)
        row = 0
        for l in range(inx + 2):
            cnt = 2 ** l
            xs[l] = u[row * be:(row + cnt) * be].reshape(cnt, be, _NF)
            row += cnt
    out_ref[...] = jnp.concatenate(xs, axis=0)


def kernel(random_vector, hlvs_pre_w0, hlvs_pre_b0, hlvs_pre_w1, hlvs_pre_b1,
           hlvs_post_w0, hlvs_post_b0, hlvs_post_w1, hlvs_post_b1,
           br_w0, br_b0, br_w1, br_b1, br_w2, br_b2,
           msg_w0, msg_b0, msg_w1, msg_b1, msg_w2, msg_b2,
           upd_w0, upd_b0, upd_w1, upd_b1, upd_w2, upd_b2):
    weights = []
    for w, b in ((hlvs_pre_w0, hlvs_pre_b0), (hlvs_pre_w1, hlvs_pre_b1),
                 (hlvs_post_w0, hlvs_post_b0), (hlvs_post_w1, hlvs_post_b1),
                 (br_w0, br_b0), (br_w1, br_b1), (br_w2, br_b2),
                 (msg_w0, msg_b0), (msg_w1, msg_b1), (msg_w2, msg_b2),
                 (upd_w0, upd_b0), (upd_w1, upd_b1), (upd_w2, upd_b2)):
        weights.append(w)
        weights.append(b.reshape(1, -1))
    nblocks = _N_EVENTS // _BE
    wspecs = [pl.BlockSpec(w.shape, lambda i: (0, 0)) for w in weights]
    out = pl.pallas_call(
        _tree_body,
        grid=(nblocks,),
        in_specs=[pl.BlockSpec((_BE, _NF), lambda i: (i, 0))] + wspecs,
        out_specs=pl.BlockSpec((_NNODES, _BE, _NF), lambda i: (0, i, 0)),
        out_shape=jax.ShapeDtypeStruct((_NNODES, _N_EVENTS, _NF),
                                       jnp.float32),
        compiler_params=pltpu.CompilerParams(
            dimension_semantics=("arbitrary",)),
    )(random_vector, *weights)
    return jnp.transpose(out[:, :, :_NF_IN], (1, 0, 2))


# speedup vs baseline: 1.2547x; 1.2547x over previous
import jax
import jax.numpy as jnp
from jax.experimental import pallas as pl
from jax.experimental.pallas import tpu as pltpu

_N_EVENTS = 10000
_NSPLITS = 5
_NF_IN = 4
_NF = 32
_NG = 16
_BE = 400
_NNODES = 2 ** (_NSPLITS + 1) - 1


def _mm(x, w, b):
    return jnp.maximum(jnp.dot(x, w, preferred_element_type=jnp.float32) + b, 0.0)


def _tree_body(rv_ref, *refs):
    out_ref = refs[-1]
    (pw0, pb0, pw1, pb1, qw0, qb0, qw1, qb1,
     bw0, bb0, bw1, bb1, bw2, bb2,
     mw0, mb0, mw1, mb1, mw2, mb2,
     uw0, ub0, uw1, ub1, uw2, ub2) = [r[...] for r in refs[:-1]]
    be = rv_ref.shape[0]
    xs = [rv_ref[...].reshape(1, be, _NF)]
    for inx in range(_NSPLITS):
        na = 2 ** (inx + 1) - 1
        xa = jnp.concatenate([xs[l].reshape(-1, _NF) for l in range(inx + 1)],
                             axis=0)
        h = _mm(_mm(xa, pw0, pb0), pw1, pb1)
        pooled = h.reshape(na, be, _NG).sum(axis=0) * (1.0 / na)
        g = _mm(_mm(pooled, qw0, qb0), qw1, qb1)
        nl = 2 ** inx
        leaf = xs[inx].reshape(nl * be, _NF)
        gl = jnp.broadcast_to(g, (nl, be, _NG)).reshape(nl * be, _NG)
        c = _mm(_mm(_mm(jnp.concatenate([leaf, gl], axis=1), bw0, bb0),
                    bw1, bb1), bw2, bb2)
        c0 = c[:, :_NF].reshape(nl, 1, be, _NF)
        c1 = c[:, _NF:].reshape(nl, 1, be, _NF)
        xs.append(jnp.concatenate([c0, c1], axis=1).reshape(2 * nl, be, _NF))
        ga = jnp.broadcast_to(g, (na, be, _NG)).reshape(na * be, _NG)
        m = _mm(_mm(_mm(jnp.concatenate([xa, ga], axis=1), mw0, mb0),
                    mw1, mb1), mw2, mb2)
        m = m.reshape(na, be, _NF)
        s_lvl = [m[0:1]]
        row = 1
        for l in range(1, inx + 1):
            cnt = 2 ** l
            s_lvl.append(jnp.repeat(s_lvl[l - 1], 2, axis=0) + m[row:row + cnt])
            row += cnt
        agg = [jnp.zeros((1, be, _NF), jnp.float32)]
        for l in range(1, inx + 2):
            agg.append(jnp.repeat(s_lvl[l - 1], 2, axis=0))
        ua = jnp.concatenate([
            jnp.concatenate(
                [xs[l], agg[l], jnp.broadcast_to(g, (2 ** l, be, _NG))], axis=2
            ).reshape(-1, 2 * _NF + _NG)
            for l in range(inx + 2)
        ], axis=0)
        u = _mm(_mm(_mm(ua, uw0, ub0), uw1, ub1), uw2, ub2)
        row = 0
        for l in range(inx + 2):
            cnt = 2 ** l
            xs[l] = u[row * be:(row + cnt) * be].reshape(cnt, be, _NF)
            row += cnt
    out_ref[...] = jnp.concatenate(
        [xs[l][:, :, :_NF_IN] for l in range(_NSPLITS + 1)], axis=0)


def kernel(random_vector, hlvs_pre_w0, hlvs_pre_b0, hlvs_pre_w1, hlvs_pre_b1,
           hlvs_post_w0, hlvs_post_b0, hlvs_post_w1, hlvs_post_b1,
           br_w0, br_b0, br_w1, br_b1, br_w2, br_b2,
           msg_w0, msg_b0, msg_w1, msg_b1, msg_w2, msg_b2,
           upd_w0, upd_b0, upd_w1, upd_b1, upd_w2, upd_b2):
    weights = []
    for w, b in ((hlvs_pre_w0, hlvs_pre_b0), (hlvs_pre_w1, hlvs_pre_b1),
                 (hlvs_post_w0, hlvs_post_b0), (hlvs_post_w1, hlvs_post_b1),
                 (br_w0, br_b0), (br_w1, br_b1), (br_w2, br_b2),
                 (msg_w0, msg_b0), (msg_w1, msg_b1), (msg_w2, msg_b2),
                 (upd_w0, upd_b0), (upd_w1, upd_b1), (upd_w2, upd_b2)):
        weights.append(w)
        weights.append(b.reshape(1, -1))
    nblocks = _N_EVENTS // _BE
    wspecs = [pl.BlockSpec(w.shape, lambda i: (0, 0)) for w in weights]
    out = pl.pallas_call(
        _tree_body,
        grid=(nblocks,),
        in_specs=[pl.BlockSpec((_BE, _NF), lambda i: (i, 0))] + wspecs,
        out_specs=pl.BlockSpec((_NNODES, _BE, _NF_IN), lambda i: (0, i, 0)),
        out_shape=jax.ShapeDtypeStruct((_NNODES, _N_EVENTS, _NF_IN),
                                       jnp.float32),
        compiler_params=pltpu.CompilerParams(
            dimension_semantics=("arbitrary",)),
    )(random_vector, *weights)
    return jnp.transpose(out, (1, 0, 2))


# speedup vs baseline: 1.4451x; 1.1517x over previous
import jax
import jax.numpy as jnp
from jax.experimental import pallas as pl
from jax.experimental.pallas import tpu as pltpu

_N_EVENTS = 10000
_NSPLITS = 5
_NF_IN = 4
_NF = 32
_NG = 16
_BE = 400
_NNODES = 2 ** (_NSPLITS + 1) - 1


def _mm(x, w, b):
    return jnp.maximum(jnp.dot(x, w, preferred_element_type=jnp.float32) + b, 0.0)


def _tree_body(rv_ref, *refs):
    out_ref = refs[-1]
    (pw0, pb0, pw1, pb1, qw0, qb0, qw1, qb1,
     bw0, bb0, bw1, bb1, bw2, bb2,
     mw0, mb0, mw1, mb1, mw2, mb2,
     uw0, ub0, uw1, ub1, uw2, ub2) = [r[...] for r in refs[:-1]]
    be = rv_ref.shape[0]
    xs = [rv_ref[...].reshape(1, be, _NF)]
    for inx in range(_NSPLITS):
        na = 2 ** (inx + 1) - 1
        xa = jnp.concatenate([xs[l].reshape(-1, _NF) for l in range(inx + 1)],
                             axis=0)
        h = _mm(_mm(xa, pw0, pb0), pw1, pb1)
        pooled = h.reshape(na, be, _NG).sum(axis=0) * (1.0 / na)
        g = _mm(_mm(pooled, qw0, qb0), qw1, qb1)
        nl = 2 ** inx
        leaf = xs[inx].reshape(nl * be, _NF)
        gl = jnp.broadcast_to(g, (nl, be, _NG)).reshape(nl * be, _NG)
        c = _mm(_mm(_mm(jnp.concatenate([leaf, gl], axis=1), bw0, bb0),
                    bw1, bb1), bw2, bb2)
        c0 = c[:, :_NF].reshape(nl, 1, be, _NF)
        c1 = c[:, _NF:].reshape(nl, 1, be, _NF)
        xs.append(jnp.concatenate([c0, c1], axis=1).reshape(2 * nl, be, _NF))
        ga = jnp.broadcast_to(g, (na, be, _NG)).reshape(na * be, _NG)
        m = _mm(_mm(_mm(jnp.concatenate([xa, ga], axis=1), mw0, mb0),
                    mw1, mb1), mw2, mb2)
        m = m.reshape(na, be, _NF)
        s_lvl = [m[0:1]]
        row = 1
        for l in range(1, inx + 1):
            cnt = 2 ** l
            s_lvl.append(jnp.repeat(s_lvl[l - 1], 2, axis=0) + m[row:row + cnt])
            row += cnt
        agg = [jnp.zeros((1, be, _NF), jnp.float32)]
        for l in range(1, inx + 2):
            agg.append(jnp.repeat(s_lvl[l - 1], 2, axis=0))
        ua = jnp.concatenate([
            jnp.concatenate(
                [xs[l], agg[l], jnp.broadcast_to(g, (2 ** l, be, _NG))], axis=2
            ).reshape(-1, 2 * _NF + _NG)
            for l in range(inx + 2)
        ], axis=0)
        u = _mm(_mm(_mm(ua, uw0, ub0), uw1, ub1), uw2, ub2)
        row = 0
        for l in range(inx + 2):
            cnt = 2 ** l
            xs[l] = u[row * be:(row + cnt) * be].reshape(cnt, be, _NF)
            row += cnt
    pieces = []
    for l in range(_NSPLITS + 1):
        for i in range(2 ** l):
            pieces.append(xs[l][i, :, :_NF_IN])
    out_ref[...] = jnp.concatenate(pieces, axis=1)


def kernel(random_vector, hlvs_pre_w0, hlvs_pre_b0, hlvs_pre_w1, hlvs_pre_b1,
           hlvs_post_w0, hlvs_post_b0, hlvs_post_w1, hlvs_post_b1,
           br_w0, br_b0, br_w1, br_b1, br_w2, br_b2,
           msg_w0, msg_b0, msg_w1, msg_b1, msg_w2, msg_b2,
           upd_w0, upd_b0, upd_w1, upd_b1, upd_w2, upd_b2):
    weights = []
    for w, b in ((hlvs_pre_w0, hlvs_pre_b0), (hlvs_pre_w1, hlvs_pre_b1),
                 (hlvs_post_w0, hlvs_post_b0), (hlvs_post_w1, hlvs_post_b1),
                 (br_w0, br_b0), (br_w1, br_b1), (br_w2, br_b2),
                 (msg_w0, msg_b0), (msg_w1, msg_b1), (msg_w2, msg_b2),
                 (upd_w0, upd_b0), (upd_w1, upd_b1), (upd_w2, upd_b2)):
        weights.append(w)
        weights.append(b.reshape(1, -1))
    nblocks = _N_EVENTS // _BE
    wspecs = [pl.BlockSpec(w.shape, lambda i: (0, 0)) for w in weights]
    out = pl.pallas_call(
        _tree_body,
        grid=(nblocks,),
        in_specs=[pl.BlockSpec((_BE, _NF), lambda i: (i, 0))] + wspecs,
        out_specs=pl.BlockSpec((_BE, _NNODES * _NF_IN), lambda i: (i, 0)),
        out_shape=jax.ShapeDtypeStruct((_N_EVENTS, _NNODES * _NF_IN),
                                       jnp.float32),
        compiler_params=pltpu.CompilerParams(
            dimension_semantics=("arbitrary",)),
    )(random_vector, *weights)
    return out.reshape(_N_EVENTS, _NNODES, _NF_IN)
